# baseline (speedup 1.0000x reference)
import functools

import jax
import jax.numpy as jnp
from jax.experimental import pallas as pl
from jax.experimental.pallas import tpu as pltpu

_D = 2048
_E = 16
_K = 2
_BLK = 1024


def _router_body(x_ref, w_ref, idx_ref, wgt_ref, logits_ref):
    x = x_ref[...]
    w = w_ref[...]
    logits = jax.lax.dot_general(
        x, w, (((1,), (1,)), ((), ())),
        preferred_element_type=jnp.float32)
    logits_ref[...] = logits
    iota = jax.lax.broadcasted_iota(jnp.int32, logits.shape, 1)
    m1 = jnp.max(logits, axis=1, keepdims=True)
    i1 = jnp.min(jnp.where(logits == m1, iota, _E), axis=1, keepdims=True)
    masked = jnp.where(iota == i1, -jnp.inf, logits)
    m2 = jnp.max(masked, axis=1, keepdims=True)
    i2 = jnp.min(jnp.where(masked == m2, iota, _E), axis=1, keepdims=True)
    e2 = jnp.exp(m2 - m1)
    denom = 1.0 + e2
    idx_ref[...] = jnp.concatenate([i1, i2], axis=1)
    wgt_ref[...] = jnp.concatenate([1.0 / denom, e2 / denom], axis=1)


@functools.partial(jax.jit, static_argnames=("interpret",))
def kernel(x, W, interpret=False):
    b, t, d = x.shape
    bt = b * t
    x2 = x.reshape(bt, d)
    grid = (bt // _BLK,)
    idx, wgt, logits = pl.pallas_call(
        _router_body,
        grid=grid,
        in_specs=[
            pl.BlockSpec((_BLK, d), lambda i: (i, 0)),
            pl.BlockSpec((_E, d), lambda i: (0, 0)),
        ],
        out_specs=[
            pl.BlockSpec((_BLK, _K), lambda i: (i, 0)),
            pl.BlockSpec((_BLK, _K), lambda i: (i, 0)),
            pl.BlockSpec((_BLK, _E), lambda i: (i, 0)),
        ],
        out_shape=[
            jax.ShapeDtypeStruct((bt, _K), jnp.int32),
            jax.ShapeDtypeStruct((bt, _K), jnp.float32),
            jax.ShapeDtypeStruct((bt, _E), jnp.float32),
        ],
        compiler_params=pltpu.CompilerParams(
            dimension_semantics=("parallel",)),
        interpret=interpret,
    )(x2, W)
    return (idx.reshape(b, t, _K),
            wgt.reshape(b, t, _K),
            logits.reshape(b, t, _E))
